# Initial kernel scaffold; baseline (speedup 1.0000x reference)
import jax
import jax.numpy as jnp
from jax.experimental import pallas as pl


def kernel(z_e, codebook):
    raise NotImplementedError("write your pallas kernel here")


# speedup vs baseline: 1.0366x; 1.0366x over previous
import functools

import jax
import jax.numpy as jnp
from jax import lax
from jax.experimental import pallas as pl

K = 8192
D = 64
BETA = 0.25

BN = 256
KB = 2048
N_TOK = 16 * 576


def _dist_argmin_body(z_ref, zn_ref, cb_ref, cn_ref, kidx_ref, loss_ref):
    i = pl.program_id(0)
    z = z_ref[...]
    zn = zn_ref[...]
    best = None
    bestidx = None
    lossval = None
    for j in range(K // KB):
        cb_chunk = cb_ref[pl.ds(j * KB, KB), :]
        m = lax.dot_general(z, cb_chunk, (((1,), (1,)), ((), ())),
                            preferred_element_type=jnp.float32)
        cn_chunk = cn_ref[:, pl.ds(j * KB, KB)]
        d = zn - 2.0 * m + cn_chunk
        mind = jnp.min(d, axis=1, keepdims=True)
        iota = lax.broadcasted_iota(jnp.int32, (BN, KB), 1) + j * KB
        midx = jnp.min(jnp.where(d == mind, iota, jnp.int32(2**30)),
                       axis=1, keepdims=True)
        if best is None:
            bestidx, lossval = midx, mind
        else:
            take = mind < best
            bestidx = jnp.where(take, midx, bestidx)
            lossval = jnp.where(take, mind, lossval)
        nb = mind if best is None else jnp.where(take, mind, best)
        best = nb.astype(jnp.bfloat16).astype(jnp.float32)
    kidx_ref[...] = bestidx

    @pl.when(i == 0)
    def _init():
        loss_ref[...] = jnp.zeros((1, 1), jnp.float32)

    loss_ref[...] += jnp.sum(lossval, keepdims=True) * (BETA / (N_TOK * D))


@functools.partial(jax.jit, static_argnums=())
def _dist_argmin(z, zn, codebook, cn):
    n = z.shape[0]
    grid = (n // BN,)
    return pl.pallas_call(
        _dist_argmin_body,
        grid=grid,
        in_specs=[
            pl.BlockSpec((BN, D), lambda i: (i, 0)),
            pl.BlockSpec((BN, 1), lambda i: (i, 0)),
            pl.BlockSpec((K, D), lambda i: (0, 0)),
            pl.BlockSpec((1, K), lambda i: (0, 0)),
        ],
        out_specs=[
            pl.BlockSpec((BN, 1), lambda i: (i, 0)),
            pl.BlockSpec((1, 1), lambda i: (0, 0)),
        ],
        out_shape=[
            jax.ShapeDtypeStruct((n, 1), jnp.int32),
            jax.ShapeDtypeStruct((1, 1), jnp.float32),
        ],
    )(z, zn, codebook, cn)


def kernel(z_e, codebook):
    B, T, Dd = z_e.shape
    z = z_e.reshape(B * T, Dd)
    zn = jnp.sum(z ** 2, axis=1, keepdims=True)
    cn = jnp.sum(codebook ** 2, axis=1).reshape(1, K)
    kidx2, loss = _dist_argmin(z, zn, codebook, cn)
    k = kidx2[:, 0]
    z_q = jnp.take(codebook, k, axis=0)
    z_q_st = z + lax.stop_gradient(z_q - z)
    counts = jnp.bincount(k, length=K)
    utilization = jnp.sum(counts > 0).astype(jnp.float32) / K
    return (z_q_st.reshape(B, T, Dd), k.reshape(B, T), loss[0, 0], utilization)


# speedup vs baseline: 1.0476x; 1.0106x over previous
import functools

import jax
import jax.numpy as jnp
from jax import lax
from jax.experimental import pallas as pl

K = 8192
D = 64
BETA = 0.25

BN = 256
KB = 2048
N_TOK = 16 * 576


def _dist_argmin_body(z_ref, zn_ref, cb_ref, cn_ref, kidx_ref, loss_ref):
    i = pl.program_id(0)
    z = z_ref[...]
    zn = zn_ref[...]
    best = None
    bestidx = None
    lossval = None
    iota = lax.broadcasted_iota(jnp.int32, (BN, KB), 1)
    for j in range(K // KB):
        cb_chunk = cb_ref[pl.ds(j * KB, KB), :]
        m = lax.dot_general(z, cb_chunk, (((1,), (1,)), ((), ())),
                            preferred_element_type=jnp.float32)
        cn_chunk = cn_ref[:, pl.ds(j * KB, KB)]
        d = (zn + m) + cn_chunk
        mind = jnp.min(d, axis=1, keepdims=True)
        midx = jnp.min(jnp.where(d == mind, iota, jnp.int32(2**30)),
                       axis=1, keepdims=True) + (j * KB)
        if best is None:
            bestidx, lossval = midx, mind
        else:
            take = mind < best
            bestidx = jnp.where(take, midx, bestidx)
            lossval = jnp.where(take, mind, lossval)
        nb = mind if best is None else jnp.where(take, mind, best)
        best = nb.astype(jnp.bfloat16).astype(jnp.float32)
    kidx_ref[...] = bestidx

    @pl.when(i == 0)
    def _init():
        loss_ref[...] = jnp.zeros((1, 1), jnp.float32)

    loss_ref[...] += jnp.sum(lossval, keepdims=True) * (BETA / (N_TOK * D))


@functools.partial(jax.jit, static_argnums=())
def _dist_argmin(z, zn, codebook, cn):
    n = z.shape[0]
    grid = (n // BN,)
    return pl.pallas_call(
        _dist_argmin_body,
        grid=grid,
        in_specs=[
            pl.BlockSpec((BN, D), lambda i: (i, 0)),
            pl.BlockSpec((BN, 1), lambda i: (i, 0)),
            pl.BlockSpec((K, D), lambda i: (0, 0)),
            pl.BlockSpec((1, K), lambda i: (0, 0)),
        ],
        out_specs=[
            pl.BlockSpec((BN, 1), lambda i: (i, 0)),
            pl.BlockSpec((1, 1), lambda i: (0, 0)),
        ],
        out_shape=[
            jax.ShapeDtypeStruct((n, 1), jnp.int32),
            jax.ShapeDtypeStruct((1, 1), jnp.float32),
        ],
    )(z, zn, codebook, cn)


def kernel(z_e, codebook):
    B, T, Dd = z_e.shape
    z = z_e.reshape(B * T, Dd)
    zn = jnp.sum(z ** 2, axis=1, keepdims=True)
    cn = jnp.sum(codebook ** 2, axis=1).reshape(1, K)
    cm2 = -2.0 * codebook
    kidx2, loss = _dist_argmin(z, zn, cm2, cn)
    k = kidx2[:, 0]
    z_q = jnp.take(codebook, k, axis=0)
    z_q_st = z + lax.stop_gradient(z_q - z)
    counts = jnp.bincount(k, length=K)
    utilization = jnp.sum(counts > 0).astype(jnp.float32) / K
    return (z_q_st.reshape(B, T, Dd), k.reshape(B, T), loss[0, 0], utilization)
